# baseline (speedup 1.0000x reference)
import functools

import jax
import jax.numpy as jnp
from jax import lax
from jax.experimental import pallas as pl
from jax.experimental.pallas import tpu as pltpu
from jax.experimental.pallas import tpu_sc as plsc

EMB = 64
CTX = 50
NCLS = 6
CB = 8


def _make_pool_kernel(batch):
    info = plsc.get_sparse_core_info()
    nw = info.num_cores * info.num_subcores
    bpw = batch // nw
    rows = CB * CTX
    nchunk = bpw // CB
    mesh = plsc.VectorSubcoreMesh(core_axis_name="c", subcore_axis_name="s")

    @functools.partial(
        pl.kernel,
        out_type=jax.ShapeDtypeStruct((batch, EMB), jnp.float32),
        mesh=mesh,
        scratch_types=[
            pltpu.VMEM((bpw * CTX,), jnp.int32),
            pltpu.VMEM((rows, EMB), jnp.float32),
            pltpu.VMEM((bpw, EMB), jnp.float32),
            pltpu.SemaphoreType.DMA,
        ],
        compiler_params=pltpu.CompilerParams(use_tc_tiling_on_sc=False),
    )
    def pool(table_hbm, idx_hbm, out_hbm, idx_v, rows_v, pooled_v, sem):
        wid = lax.axis_index("s") * info.num_cores + lax.axis_index("c")
        base = wid * bpw
        pltpu.sync_copy(idx_hbm.at[pl.ds(base * CTX, bpw * CTX)], idx_v)

        @pl.loop(0, nchunk)
        def _chunk(c):
            idx_slice = idx_v.at[pl.ds(c * rows, rows)]
            pltpu.async_copy(table_hbm.at[idx_slice], rows_v, sem).wait()
            for e in range(CB):
                accs = [rows_v[e * CTX, pl.ds(16 * k, 16)] for k in range(4)]
                for r in range(1, CTX):
                    for k in range(4):
                        accs[k] = accs[k] + rows_v[e * CTX + r, pl.ds(16 * k, 16)]
                for k in range(4):
                    pooled_v[c * CB + e, pl.ds(16 * k, 16)] = accs[k]

        pltpu.sync_copy(pooled_v, out_hbm.at[pl.ds(base, bpw)])

    return pool


def _linear_body(p_ref, wt_ref, b_ref, o_ref):
    acc = jnp.dot(p_ref[...], wt_ref[...], preferred_element_type=jnp.float32)
    o_ref[...] = jax.nn.sigmoid(acc + b_ref[...])


def _linear(pooled, wt8, b8):
    batch = pooled.shape[0]
    blk = 2048
    grid = batch // blk
    return pl.pallas_call(
        _linear_body,
        grid=(grid,),
        in_specs=[
            pl.BlockSpec((blk, EMB), lambda i: (i, 0)),
            pl.BlockSpec((EMB, 8), lambda i: (0, 0)),
            pl.BlockSpec((1, 8), lambda i: (0, 0)),
        ],
        out_specs=pl.BlockSpec((blk, 8), lambda i: (i, 0)),
        out_shape=jax.ShapeDtypeStruct((batch, 8), jnp.float32),
    )(pooled, wt8, b8)


def kernel(inputs, table, W, b):
    ctx, batch = inputs.shape
    idx_flat = inputs.T.reshape(-1).astype(jnp.int32)
    pooled = _make_pool_kernel(batch)(table, idx_flat)
    wt8 = jnp.zeros((EMB, 8), jnp.float32).at[:, :NCLS].set(W.T)
    b8 = jnp.zeros((1, 8), jnp.float32).at[0, :NCLS].set(b)
    out8 = _linear(pooled, wt8, b8)
    return out8[:, :NCLS]


# speedup vs baseline: 1.2156x; 1.2156x over previous
import functools

import jax
import jax.numpy as jnp
from jax import lax
from jax.experimental import pallas as pl
from jax.experimental.pallas import tpu as pltpu
from jax.experimental.pallas import tpu_sc as plsc

EMB = 64
CTX = 50
NCLS = 6
CB = 8
TC_VB = 4096


def _transpose_body(tt_ref, out_ref):
    t3 = tt_ref[...].T.reshape(TC_VB // 2, 2, EMB)
    out_ref[:, 0:EMB] = t3[:, 0, :]
    out_ref[:, EMB:2 * EMB] = t3[:, 1, :]


def _repack_table(table_t):
    emb, vocab = table_t.shape
    grid = (vocab + TC_VB - 1) // TC_VB
    return pl.pallas_call(
        _transpose_body,
        grid=(grid,),
        in_specs=[pl.BlockSpec((emb, TC_VB), lambda i: (0, i))],
        out_specs=pl.BlockSpec((TC_VB // 2, 2 * emb), lambda i: (i, 0)),
        out_shape=jax.ShapeDtypeStruct((vocab // 2, 2 * emb), jnp.float32),
    )(table_t)


def _make_pool_kernel(batch):
    info = plsc.get_sparse_core_info()
    nw = info.num_cores * info.num_subcores
    bpw = batch // nw
    rows = CB * CTX
    nchunk = bpw // CB
    mesh = plsc.VectorSubcoreMesh(core_axis_name="c", subcore_axis_name="s")

    @functools.partial(
        pl.kernel,
        out_type=jax.ShapeDtypeStruct((batch, EMB), jnp.float32),
        mesh=mesh,
        scratch_types=[
            pltpu.VMEM((bpw * CTX,), jnp.int32),
            pltpu.VMEM((rows, EMB), jnp.float32),
            pltpu.VMEM((bpw, EMB), jnp.float32),
            pltpu.SemaphoreType.DMA,
        ],
        compiler_params=pltpu.CompilerParams(use_tc_tiling_on_sc=False),
    )
    def pool(table_hbm, idx_hbm, out_hbm, idx_v, rows_v, pooled_v, sem):
        wid = lax.axis_index("s") * info.num_cores + lax.axis_index("c")
        base = wid * bpw
        pltpu.sync_copy(idx_hbm.at[pl.ds(base * CTX, bpw * CTX)], idx_v)

        @pl.loop(0, nchunk)
        def _chunk(c):
            idx_slice = idx_v.at[pl.ds(c * rows, rows)]
            pltpu.async_copy(table_hbm.at[idx_slice], rows_v, sem).wait()
            for e in range(CB):
                accs = [rows_v[e * CTX, pl.ds(16 * k, 16)] for k in range(4)]
                for r in range(1, CTX):
                    for k in range(4):
                        accs[k] = accs[k] + rows_v[e * CTX + r, pl.ds(16 * k, 16)]
                for k in range(4):
                    pooled_v[c * CB + e, pl.ds(16 * k, 16)] = accs[k]

        pltpu.sync_copy(pooled_v, out_hbm.at[pl.ds(base, bpw)])

    return pool


def _linear_body(p_ref, wt_ref, b_ref, o_ref):
    acc = jnp.dot(p_ref[...], wt_ref[...], preferred_element_type=jnp.float32)
    o_ref[...] = jax.nn.sigmoid(acc + b_ref[...])


def _linear(pooled, wt8, b8):
    batch = pooled.shape[0]
    blk = 2048
    grid = batch // blk
    return pl.pallas_call(
        _linear_body,
        grid=(grid,),
        in_specs=[
            pl.BlockSpec((blk, EMB), lambda i: (i, 0)),
            pl.BlockSpec((EMB, 8), lambda i: (0, 0)),
            pl.BlockSpec((1, 8), lambda i: (0, 0)),
        ],
        out_specs=pl.BlockSpec((blk, 8), lambda i: (i, 0)),
        out_shape=jax.ShapeDtypeStruct((batch, 8), jnp.float32),
    )(pooled, wt8, b8)


def kernel(inputs, table, W, b):
    ctx, batch = inputs.shape
    vocab = table.shape[0]
    idx_flat = inputs.T.reshape(-1).astype(jnp.int32)
    table_rm = _repack_table(table.T).reshape(vocab, EMB)
    pooled = _make_pool_kernel(batch)(table_rm, idx_flat)
    wt8 = jnp.zeros((EMB, 8), jnp.float32).at[:, :NCLS].set(W.T)
    b8 = jnp.zeros((1, 8), jnp.float32).at[0, :NCLS].set(b)
    out8 = _linear(pooled, wt8, b8)
    return out8[:, :NCLS]


# speedup vs baseline: 1.6195x; 1.3323x over previous
import functools

import jax
import jax.numpy as jnp
from jax import lax
from jax.experimental import pallas as pl
from jax.experimental.pallas import tpu as pltpu
from jax.experimental.pallas import tpu_sc as plsc

EMB = 64
CTX = 50
NCLS = 6
CB = 8
TC_VB = 8192


HALF = 524288


def _transpose_body(left_ref, right_ref, out_ref):
    ident = (lax.broadcasted_iota(jnp.int32, (EMB, EMB), 0)
             == lax.broadcasted_iota(jnp.int32, (EMB, EMB), 1)).astype(jnp.float32)
    dn = (((0,), (0,)), ((), ()))
    tl = lax.dot_general(left_ref[...], ident, dn,
                         preferred_element_type=jnp.float32)
    tr = lax.dot_general(right_ref[...], ident, dn,
                         preferred_element_type=jnp.float32)
    out_ref[...] = jnp.concatenate([tl, tr], axis=1)


def _repack_table(table_t):
    emb, vocab = table_t.shape
    steps = HALF // TC_VB
    max_blk = (vocab - 1) // TC_VB

    return pl.pallas_call(
        _transpose_body,
        grid=(steps,),
        in_specs=[
            pl.BlockSpec((emb, TC_VB), lambda j: (0, j)),
            pl.BlockSpec((emb, TC_VB),
                         lambda j: (0, jnp.minimum(steps + j, max_blk))),
        ],
        out_specs=pl.BlockSpec((TC_VB, 2 * emb), lambda j: (j, 0)),
        out_shape=jax.ShapeDtypeStruct((HALF, 2 * emb), jnp.float32),
    )(table_t, table_t)


def _make_pool_kernel(batch):
    info = plsc.get_sparse_core_info()
    nw = info.num_cores * info.num_subcores
    bpw = batch // nw
    rows = CB * CTX
    nchunk = bpw // CB
    mesh = plsc.VectorSubcoreMesh(core_axis_name="c", subcore_axis_name="s")

    @functools.partial(
        pl.kernel,
        out_type=jax.ShapeDtypeStruct((batch, EMB), jnp.float32),
        mesh=mesh,
        scratch_types=[
            pltpu.VMEM((bpw * CTX,), jnp.int32),
            pltpu.VMEM((rows, EMB), jnp.float32),
            pltpu.VMEM((bpw, EMB), jnp.float32),
            pltpu.SemaphoreType.DMA,
        ],
        compiler_params=pltpu.CompilerParams(use_tc_tiling_on_sc=False),
    )
    def pool(table_hbm, idx_hbm, out_hbm, idx_v, rows_v, pooled_v, sem):
        wid = lax.axis_index("s") * info.num_cores + lax.axis_index("c")
        base = wid * bpw
        pltpu.sync_copy(idx_hbm.at[pl.ds(base * CTX, bpw * CTX)], idx_v)

        @pl.loop(0, nchunk)
        def _chunk(c):
            idx_slice = idx_v.at[pl.ds(c * rows, rows)]
            pltpu.async_copy(table_hbm.at[idx_slice], rows_v, sem).wait()
            for e in range(CB):
                accs = [rows_v[e * CTX, pl.ds(16 * k, 16)] for k in range(4)]
                for r in range(1, CTX):
                    for k in range(4):
                        accs[k] = accs[k] + rows_v[e * CTX + r, pl.ds(16 * k, 16)]
                for k in range(4):
                    pooled_v[c * CB + e, pl.ds(16 * k, 16)] = accs[k]

        pltpu.sync_copy(pooled_v, out_hbm.at[pl.ds(base, bpw)])

    return pool


def _linear_body(p_ref, wt_ref, b_ref, o_ref):
    acc = jnp.dot(p_ref[...], wt_ref[...], preferred_element_type=jnp.float32)
    o_ref[...] = jax.nn.sigmoid(acc + b_ref[...])


def _linear(pooled, wt8, b8):
    batch = pooled.shape[0]
    blk = 2048
    grid = batch // blk
    return pl.pallas_call(
        _linear_body,
        grid=(grid,),
        in_specs=[
            pl.BlockSpec((blk, EMB), lambda i: (i, 0)),
            pl.BlockSpec((EMB, 8), lambda i: (0, 0)),
            pl.BlockSpec((1, 8), lambda i: (0, 0)),
        ],
        out_specs=pl.BlockSpec((blk, 8), lambda i: (i, 0)),
        out_shape=jax.ShapeDtypeStruct((batch, 8), jnp.float32),
    )(pooled, wt8, b8)


def kernel(inputs, table, W, b):
    ctx, batch = inputs.shape
    vocab = table.shape[0]
    idx_flat = inputs.T.reshape(-1).astype(jnp.int32)
    idx_flat = jnp.where(idx_flat < HALF, 2 * idx_flat, 2 * (idx_flat - HALF) + 1)
    table_rm = _repack_table(table.T).reshape(2 * HALF, EMB)
    pooled = _make_pool_kernel(batch)(table_rm, idx_flat)
    wt8 = jnp.zeros((EMB, 8), jnp.float32).at[:, :NCLS].set(W.T)
    b8 = jnp.zeros((1, 8), jnp.float32).at[0, :NCLS].set(b)
    out8 = _linear(pooled, wt8, b8)
    return out8[:, :NCLS]


# speedup vs baseline: 1.9156x; 1.1828x over previous
import functools

import jax
import jax.numpy as jnp
from jax import lax
from jax.experimental import pallas as pl
from jax.experimental.pallas import tpu as pltpu
from jax.experimental.pallas import tpu_sc as plsc

EMB = 64
CTX = 50
NCLS = 6
CB = 4
TC_VB = 8192


HALF = 524288


def _transpose_body(left_ref, right_ref, out_ref):
    ident = (lax.broadcasted_iota(jnp.int32, (EMB, EMB), 0)
             == lax.broadcasted_iota(jnp.int32, (EMB, EMB), 1)).astype(jnp.float32)
    dn = (((0,), (0,)), ((), ()))
    tl = lax.dot_general(left_ref[...], ident, dn,
                         preferred_element_type=jnp.float32)
    tr = lax.dot_general(right_ref[...], ident, dn,
                         preferred_element_type=jnp.float32)
    out_ref[...] = jnp.concatenate([tl, tr], axis=1)


def _repack_table(table_t):
    emb, vocab = table_t.shape
    steps = HALF // TC_VB
    max_blk = (vocab - 1) // TC_VB

    return pl.pallas_call(
        _transpose_body,
        grid=(steps,),
        in_specs=[
            pl.BlockSpec((emb, TC_VB), lambda j: (0, j)),
            pl.BlockSpec((emb, TC_VB),
                         lambda j: (0, jnp.minimum(steps + j, max_blk))),
        ],
        out_specs=pl.BlockSpec((TC_VB, 2 * emb), lambda j: (j, 0)),
        out_shape=jax.ShapeDtypeStruct((HALF, 2 * emb), jnp.float32),
    )(table_t, table_t)


def _make_pool_kernel(batch):
    info = plsc.get_sparse_core_info()
    nw = info.num_cores * info.num_subcores
    bpw = batch // nw
    rows = CB * CTX
    nchunk = bpw // CB
    mesh = plsc.VectorSubcoreMesh(core_axis_name="c", subcore_axis_name="s")

    @functools.partial(
        pl.kernel,
        out_type=jax.ShapeDtypeStruct((batch, EMB), jnp.float32),
        mesh=mesh,
        scratch_types=[
            pltpu.VMEM((bpw * CTX,), jnp.int32),
            pltpu.VMEM((2, rows, EMB), jnp.float32),
            pltpu.VMEM((bpw, EMB), jnp.float32),
            pltpu.SemaphoreType.DMA,
            pltpu.SemaphoreType.DMA,
        ],
        compiler_params=pltpu.CompilerParams(use_tc_tiling_on_sc=False),
    )
    def pool(table_hbm, idx_hbm, out_hbm, idx_v, rows_v, pooled_v, sem0, sem1):
        wid = lax.axis_index("s") * info.num_cores + lax.axis_index("c")
        base = wid * bpw
        pltpu.sync_copy(idx_hbm.at[pl.ds(base * CTX, bpw * CTX)], idx_v)
        sems = (sem0, sem1)

        def gather(c, b):
            return pltpu.make_async_copy(
                table_hbm.at[idx_v.at[pl.ds(c * rows, rows)]],
                rows_v.at[b], sems[b])

        gather(0, 0).start()
        gather(1, 1).start()

        @pl.loop(0, nchunk // 2)
        def _pair(i):
            for b in range(2):
                c = 2 * i + b
                gather(c, b).wait()
                for e in range(CB):
                    accs = [rows_v[b, e * CTX, pl.ds(16 * k, 16)] for k in range(4)]
                    for r in range(1, CTX):
                        for k in range(4):
                            accs[k] = accs[k] + rows_v[b, e * CTX + r, pl.ds(16 * k, 16)]
                    for k in range(4):
                        pooled_v[c * CB + e, pl.ds(16 * k, 16)] = accs[k]

                @pl.when(c + 2 < nchunk)
                def _prefetch():
                    gather(c + 2, b).start()

        pltpu.sync_copy(pooled_v, out_hbm.at[pl.ds(base, bpw)])

    return pool


def _linear_body(p_ref, wt_ref, b_ref, o_ref):
    acc = jnp.dot(p_ref[...], wt_ref[...], preferred_element_type=jnp.float32)
    o_ref[...] = jax.nn.sigmoid(acc + b_ref[...])


def _linear(pooled, wt8, b8):
    batch = pooled.shape[0]
    blk = 2048
    grid = batch // blk
    return pl.pallas_call(
        _linear_body,
        grid=(grid,),
        in_specs=[
            pl.BlockSpec((blk, EMB), lambda i: (i, 0)),
            pl.BlockSpec((EMB, 8), lambda i: (0, 0)),
            pl.BlockSpec((1, 8), lambda i: (0, 0)),
        ],
        out_specs=pl.BlockSpec((blk, 8), lambda i: (i, 0)),
        out_shape=jax.ShapeDtypeStruct((batch, 8), jnp.float32),
    )(pooled, wt8, b8)


def kernel(inputs, table, W, b):
    ctx, batch = inputs.shape
    vocab = table.shape[0]
    idx_flat = inputs.T.reshape(-1).astype(jnp.int32)
    idx_flat = jnp.where(idx_flat < HALF, 2 * idx_flat, 2 * (idx_flat - HALF) + 1)
    table_rm = _repack_table(table.T).reshape(2 * HALF, EMB)
    pooled = _make_pool_kernel(batch)(table_rm, idx_flat)
    wt8 = jnp.zeros((EMB, 8), jnp.float32).at[:, :NCLS].set(W.T)
    b8 = jnp.zeros((1, 8), jnp.float32).at[0, :NCLS].set(b)
    out8 = _linear(pooled, wt8, b8)
    return out8[:, :NCLS]
